# Initial kernel scaffold; baseline (speedup 1.0000x reference)
import jax
import jax.numpy as jnp
from jax.experimental import pallas as pl


def kernel(z, lid_W, lid_b, bbd1_W, bbd1_b, bbd2_W, bbd2_b, ed1_W, ed1_b, ed2_W, ed2_b, mp1_W, mp1_b, mp2_W, mp2_b, mp3_W, mp3_b, ned_W, ned_b, lh_W, lh_b, bh_W, bh_b, bdh_W, bdh_b, mh_W, mh_b):
    raise NotImplementedError("write your pallas kernel here")


# speedup vs baseline: 1.1315x; 1.1315x over previous
import jax
import jax.numpy as jnp
from jax import lax
from jax.experimental import pallas as pl

B = 64
NB = 30
NSEM = 11
D = 256
F32 = jnp.float32


def _lot_body(z_ref, w_ref, b_ref, out_ref):
    acc = jnp.dot(z_ref[:, :], w_ref[:, :], preferred_element_type=F32)
    out_ref[:, :] = jnp.maximum(acc + b_ref[:, :], 0.0)


def _proj_body(lot_ref, bw1_ref, bb1_ref, bw2_ref, bb2_ref,
               w1_ref, p1_ref, w2_ref, p2_ref,
               ar_ref, a_ref, b2_ref):
    lot = lot_ref[:, :]
    h = jnp.maximum(jnp.dot(lot, bw1_ref[:, :], preferred_element_type=F32)
                    + bb1_ref[:, :], 0.0)
    ar_ref[:, :] = jnp.dot(h, bw2_ref[:, :], preferred_element_type=F32) + bb2_ref[:, :]
    a_ref[:, :] = jnp.dot(lot, w1_ref[:, :], preferred_element_type=F32) + p1_ref[:, :]
    b2_ref[:, :] = jnp.dot(lot, w2_ref[:, :], preferred_element_type=F32) + p2_ref[:, :]


def _adj_body(a_ref, bt_ref, lot_ref, w_ref, eb_ref, adj_ref, s_ref, nbr_ref):
    b = pl.program_id(0)
    A = a_ref[0, :, :]
    Bt = bt_ref[0, :, :]
    T = jnp.maximum(A[:, None, :] + Bt[None, :, :], 0.0)
    logit = jnp.sum(T * w_ref[:, :][None, :, :], axis=-1) + eb_ref[0, 0]
    adjb = jax.nn.sigmoid(logit)
    adj_ref[0, :, :] = adjb
    af = (adjb >= 0.5).astype(F32)
    nbr = lax.dot_general(af, lot_ref[0, :, :], (((0,), (0,)), ((), ())),
                          preferred_element_type=F32)

    @pl.when(b == 0)
    def _init():
        s_ref[:, :] = af
        nbr_ref[:, :] = nbr

    @pl.when(b > 0)
    def _acc():
        s_ref[:, :] = s_ref[:, :] + af
        nbr_ref[:, :] = nbr_ref[:, :] + nbr


def _mp_body(s_ref, nbr_ref, adj0_ref, x30_ref,
             w1il_ref, w1ip_ref, w1jl_ref, w1jp_ref, b1_ref,
             w2i_ref, w2j_ref, b2_ref, w3i_ref, w3j_ref, b3_ref,
             nedw_ref, nedb_ref, lhw_ref, lhb_ref, bhw_ref, bhb_ref,
             bdw_ref, bdb_ref, mhw_ref, mhb_ref,
             lep_ref, lg_ref, lu_ref, lb_ref, lm_ref):
    S = s_ref[:, :]
    ones = jnp.ones((NB, 1), F32)
    deg = lax.dot_general(S, ones, (((0,), (0,)), ((), ())),
                          preferred_element_type=F32)
    invd = 1.0 / jnp.where(deg > 0, deg, 1.0)
    mask = deg > 0

    nbrm = nbr_ref[:, :] * invd
    sm = lax.dot_general(S, w1jp_ref[:, :], (((0,), (0,)), ((), ())),
                         preferred_element_type=F32) * invd
    out1 = (jnp.dot(x30_ref[:, :], w1il_ref[:, :], preferred_element_type=F32)
            + w1ip_ref[:, :]
            + jnp.dot(nbrm, w1jl_ref[:, :], preferred_element_type=F32)
            + sm + b1_ref[:, :])
    d = jnp.maximum(jnp.where(mask, out1, 0.0), 0.0)

    af0 = (adj0_ref[:, :] >= 0.5).astype(F32)
    for wi_ref, wj_ref, bb_ref in ((w2i_ref, w2j_ref, b2_ref),
                                   (w3i_ref, w3j_ref, b3_ref)):
        nbr = lax.dot_general(af0, d, (((0,), (0,)), ((), ())),
                              preferred_element_type=F32) * invd
        out = (jnp.dot(d, wi_ref[:, :], preferred_element_type=F32)
               + jnp.dot(nbr, wj_ref[:, :], preferred_element_type=F32)
               + bb_ref[:, :])
        d = jnp.maximum(jnp.where(mask, out, 0.0), 0.0)

    def head(w_ref, b_ref):
        return jnp.dot(d, w_ref[:, :], preferred_element_type=F32) + b_ref[:, :]

    rest = B * NB - NB
    ned = jax.nn.sigmoid(head(nedw_ref, nedb_ref))
    lep_ref[:NB, :] = ned
    lep_ref[NB:, :] = jnp.broadcast_to(jax.nn.sigmoid(nedb_ref[:, :]), (rest, 1))

    def softmax(x):
        m = jnp.max(x, axis=-1, keepdims=True)
        e = jnp.exp(x - m)
        return e / jnp.sum(e, axis=-1, keepdims=True)

    lu_ref[:NB, :] = softmax(head(lhw_ref, lhb_ref))
    lu_ref[NB:, :] = jnp.broadcast_to(softmax(lhb_ref[:, :]), (rest, NSEM))

    lg_ref[:NB, :] = head(bhw_ref, bhb_ref)
    lg_ref[NB:, :] = jnp.broadcast_to(bhb_ref[:, :], (rest, 5))
    lb_ref[:NB, :] = head(bdw_ref, bdb_ref)
    lb_ref[NB:, :] = jnp.broadcast_to(bdb_ref[:, :], (rest, 4))
    lm_ref[:NB, :] = head(mhw_ref, mhb_ref)
    lm_ref[NB:, :] = jnp.broadcast_to(mhb_ref[:, :], (rest, 2))


def kernel(z, lid_W, lid_b, bbd1_W, bbd1_b, bbd2_W, bbd2_b, ed1_W, ed1_b,
           ed2_W, ed2_b, mp1_W, mp1_b, mp2_W, mp2_b, mp3_W, mp3_b,
           ned_W, ned_b, lh_W, lh_b, bh_W, bh_b, bdh_W, bdh_b, mh_W, mh_b):
    NBLK = 6
    BN = (NB * D) // NBLK
    lot2d = pl.pallas_call(
        _lot_body,
        grid=(NBLK,),
        in_specs=[
            pl.BlockSpec((B, D), lambda j: (0, 0)),
            pl.BlockSpec((D, BN), lambda j: (0, j)),
            pl.BlockSpec((1, BN), lambda j: (0, j)),
        ],
        out_specs=pl.BlockSpec((B, BN), lambda j: (0, j)),
        out_shape=jax.ShapeDtypeStruct((B, NB * D), F32),
    )(z, lid_W, lid_b.reshape(1, NB * D))
    lot = lot2d.reshape(B * NB, D)

    W1l, W1p = ed1_W[:D], ed1_W[D:D + NB]
    W2l, W2p = ed1_W[D + NB:2 * D + NB], ed1_W[2 * D + NB:]
    RB = 240
    pos1 = jnp.tile(W1p, (RB // NB, 1)) + ed1_b[None, :]
    pos2 = jnp.tile(W2p, (RB // NB, 1))
    ar, A2d, B2d = pl.pallas_call(
        _proj_body,
        grid=(B * NB // RB,),
        in_specs=[
            pl.BlockSpec((RB, D), lambda j: (j, 0)),
            pl.BlockSpec((D, D), lambda j: (0, 0)),
            pl.BlockSpec((1, D), lambda j: (0, 0)),
            pl.BlockSpec((D, 1), lambda j: (0, 0)),
            pl.BlockSpec((1, 1), lambda j: (0, 0)),
            pl.BlockSpec((D, D), lambda j: (0, 0)),
            pl.BlockSpec((RB, D), lambda j: (0, 0)),
            pl.BlockSpec((D, D), lambda j: (0, 0)),
            pl.BlockSpec((RB, D), lambda j: (0, 0)),
        ],
        out_specs=[
            pl.BlockSpec((RB, 1), lambda j: (j, 0)),
            pl.BlockSpec((RB, D), lambda j: (j, 0)),
            pl.BlockSpec((RB, D), lambda j: (j, 0)),
        ],
        out_shape=[
            jax.ShapeDtypeStruct((B * NB, 1), F32),
            jax.ShapeDtypeStruct((B * NB, D), F32),
            jax.ShapeDtypeStruct((B * NB, D), F32),
        ],
    )(lot, bbd1_W, bbd1_b.reshape(1, D), bbd2_W, bbd2_b.reshape(1, 1),
      W1l, pos1, W2l, pos2)

    A3 = A2d.reshape(B, NB, D)
    B3 = B2d.reshape(B, NB, D)
    lot3 = lot.reshape(B, NB, D)
    adj, S, nbr = pl.pallas_call(
        _adj_body,
        grid=(B,),
        in_specs=[
            pl.BlockSpec((1, NB, D), lambda b: (b, 0, 0)),
            pl.BlockSpec((1, NB, D), lambda b: (b, 0, 0)),
            pl.BlockSpec((1, NB, D), lambda b: (b, 0, 0)),
            pl.BlockSpec((1, D), lambda b: (0, 0)),
            pl.BlockSpec((1, 1), lambda b: (0, 0)),
        ],
        out_specs=[
            pl.BlockSpec((1, NB, NB), lambda b: (b, 0, 0)),
            pl.BlockSpec((NB, NB), lambda b: (0, 0)),
            pl.BlockSpec((NB, D), lambda b: (0, 0)),
        ],
        out_shape=[
            jax.ShapeDtypeStruct((B, NB, NB), F32),
            jax.ShapeDtypeStruct((NB, NB), F32),
            jax.ShapeDtypeStruct((NB, D), F32),
        ],
    )(A3, B3, lot3, ed2_W.reshape(1, D), ed2_b.reshape(1, 1))

    full = lambda *s: pl.BlockSpec(s, lambda: tuple(0 for _ in s))
    lep, lg, lu, lb, lm = pl.pallas_call(
        _mp_body,
        in_specs=[
            full(NB, NB), full(NB, D), full(NB, NB),
            full(NB, D),
            full(D, D), full(NB, D), full(D, D), full(NB, D), full(1, D),
            full(D, D), full(D, D), full(1, D),
            full(D, D), full(D, D), full(1, D),
            full(D, 1), full(1, 1), full(D, NSEM), full(1, NSEM),
            full(D, 5), full(1, 5), full(D, 4), full(1, 4),
            full(D, 2), full(1, 2),
        ],
        out_specs=[full(B * NB, 1), full(B * NB, 5), full(B * NB, NSEM),
                   full(B * NB, 4), full(B * NB, 2)],
        out_shape=[
            jax.ShapeDtypeStruct((B * NB, 1), F32),
            jax.ShapeDtypeStruct((B * NB, 5), F32),
            jax.ShapeDtypeStruct((B * NB, NSEM), F32),
            jax.ShapeDtypeStruct((B * NB, 4), F32),
            jax.ShapeDtypeStruct((B * NB, 2), F32),
        ],
    )(S, nbr, adj[0], lot[:NB],
      mp1_W[:D], mp1_W[D:D + NB], mp1_W[D + NB:2 * D + NB], mp1_W[2 * D + NB:],
      mp1_b.reshape(1, D),
      mp2_W[:D], mp2_W[D:], mp2_b.reshape(1, D),
      mp3_W[:D], mp3_W[D:], mp3_b.reshape(1, D),
      ned_W, ned_b.reshape(1, 1), lh_W, lh_b.reshape(1, NSEM),
      bh_W, bh_b.reshape(1, 5), bdh_W, bdh_b.reshape(1, 4),
      mh_W, mh_b.reshape(1, 2))

    return (lep, lg, lu, lb, lm, adj, ar)


# speedup vs baseline: 1.4300x; 1.2637x over previous
import jax
import jax.numpy as jnp
from jax import lax
from jax.experimental import pallas as pl
from jax.experimental.pallas import tpu as pltpu

B = 64
NB = 30
NSEM = 11
D = 256
GB = 8
RB = GB * NB
F32 = jnp.float32


def _lot_body(z_ref, w_ref, b_ref, out_ref):
    acc = jnp.dot(z_ref[:, :], w_ref[:, :], preferred_element_type=F32)
    out_ref[:, :] = jnp.maximum(acc + b_ref[:, :], 0.0)


def _fused_body(lot2_ref, lot3_ref,
                bw1_ref, bb1_ref, bw2_ref, bb2_ref,
                w1_ref, p1_ref, w2_ref, p2_ref, edw_ref, edb_ref,
                w1il_ref, w1ip_ref, w1jl_ref, w1jp_ref, b1_ref,
                w2i_ref, w2j_ref, b2_ref, w3i_ref, w3j_ref, b3_ref,
                nedw_ref, nedb_ref, lhw_ref, lhb_ref, bhw_ref, bhb_ref,
                bdw_ref, bdb_ref, mhw_ref, mhb_ref,
                ar_ref, adj_ref, lep_ref, lg_ref, lu_ref, lb_ref, lm_ref,
                s_ref, nbr_ref, x30_ref, af0_ref):
    j = pl.program_id(0)
    lot = lot2_ref[:, :]

    h = jnp.maximum(jnp.dot(lot, bw1_ref[:, :], preferred_element_type=F32)
                    + bb1_ref[:, :], 0.0)
    ar_ref[:, :] = jnp.dot(h, bw2_ref[:, :], preferred_element_type=F32) + bb2_ref[:, :]

    A = jnp.dot(lot, w1_ref[:, :], preferred_element_type=F32) + p1_ref[:, :]
    Bt = jnp.dot(lot, w2_ref[:, :], preferred_element_type=F32) + p2_ref[:, :]

    w = edw_ref[:, :][None, :, :]
    eb = edb_ref[0, 0]
    s_acc = None
    nbr_acc = None
    for i in range(GB):
        Ai = A[i * NB:(i + 1) * NB, :]
        Bi = Bt[i * NB:(i + 1) * NB, :]
        T = jnp.maximum(Ai[:, None, :] + Bi[None, :, :], 0.0)
        logit = jnp.sum(T * w, axis=-1) + eb
        adjb = jax.nn.sigmoid(logit)
        adj_ref[i, :, :] = adjb
        af = (adjb >= 0.5).astype(F32)
        nbr = lax.dot_general(af, lot3_ref[i, :, :], (((0,), (0,)), ((), ())),
                              preferred_element_type=F32)
        s_acc = af if s_acc is None else s_acc + af
        nbr_acc = nbr if nbr_acc is None else nbr_acc + nbr
        if i == 0:
            @pl.when(j == 0)
            def _save0():
                x30_ref[:, :] = lot3_ref[0, :, :]
                af0_ref[:, :] = af

    @pl.when(j == 0)
    def _init():
        s_ref[:, :] = s_acc
        nbr_ref[:, :] = nbr_acc

    @pl.when(j > 0)
    def _acc():
        s_ref[:, :] = s_ref[:, :] + s_acc
        nbr_ref[:, :] = nbr_ref[:, :] + nbr_acc

    @pl.when(j == pl.num_programs(0) - 1)
    def _mp():
        S = s_ref[:, :]
        ones = jnp.ones((NB, 1), F32)
        deg = lax.dot_general(S, ones, (((0,), (0,)), ((), ())),
                              preferred_element_type=F32)
        invd = 1.0 / jnp.where(deg > 0, deg, 1.0)
        mask = deg > 0

        nbrm = nbr_ref[:, :] * invd
        sm = lax.dot_general(S, w1jp_ref[:, :], (((0,), (0,)), ((), ())),
                             preferred_element_type=F32) * invd
        out1 = (jnp.dot(x30_ref[:, :], w1il_ref[:, :], preferred_element_type=F32)
                + w1ip_ref[:, :]
                + jnp.dot(nbrm, w1jl_ref[:, :], preferred_element_type=F32)
                + sm + b1_ref[:, :])
        d = jnp.maximum(jnp.where(mask, out1, 0.0), 0.0)

        af0 = af0_ref[:, :]
        for wi_ref, wj_ref, bb_ref in ((w2i_ref, w2j_ref, b2_ref),
                                       (w3i_ref, w3j_ref, b3_ref)):
            nbr2 = lax.dot_general(af0, d, (((0,), (0,)), ((), ())),
                                   preferred_element_type=F32) * invd
            out = (jnp.dot(d, wi_ref[:, :], preferred_element_type=F32)
                   + jnp.dot(nbr2, wj_ref[:, :], preferred_element_type=F32)
                   + bb_ref[:, :])
            d = jnp.maximum(jnp.where(mask, out, 0.0), 0.0)

        def head(w_ref, b_ref):
            return jnp.dot(d, w_ref[:, :], preferred_element_type=F32) + b_ref[:, :]

        def softmax(x):
            m = jnp.max(x, axis=-1, keepdims=True)
            e = jnp.exp(x - m)
            return e / jnp.sum(e, axis=-1, keepdims=True)

        rest = B * NB - NB
        lep_ref[:NB, :] = jax.nn.sigmoid(head(nedw_ref, nedb_ref))
        lep_ref[NB:, :] = jnp.broadcast_to(jax.nn.sigmoid(nedb_ref[:, :]), (rest, 1))
        lu_ref[:NB, :] = softmax(head(lhw_ref, lhb_ref))
        lu_ref[NB:, :] = jnp.broadcast_to(softmax(lhb_ref[:, :]), (rest, NSEM))
        lg_ref[:NB, :] = head(bhw_ref, bhb_ref)
        lg_ref[NB:, :] = jnp.broadcast_to(bhb_ref[:, :], (rest, 5))
        lb_ref[:NB, :] = head(bdw_ref, bdb_ref)
        lb_ref[NB:, :] = jnp.broadcast_to(bdb_ref[:, :], (rest, 4))
        lm_ref[:NB, :] = head(mhw_ref, mhb_ref)
        lm_ref[NB:, :] = jnp.broadcast_to(mhb_ref[:, :], (rest, 2))


def kernel(z, lid_W, lid_b, bbd1_W, bbd1_b, bbd2_W, bbd2_b, ed1_W, ed1_b,
           ed2_W, ed2_b, mp1_W, mp1_b, mp2_W, mp2_b, mp3_W, mp3_b,
           ned_W, ned_b, lh_W, lh_b, bh_W, bh_b, bdh_W, bdh_b, mh_W, mh_b):
    NBLK = 6
    BN = (NB * D) // NBLK
    lot2d = pl.pallas_call(
        _lot_body,
        grid=(NBLK,),
        in_specs=[
            pl.BlockSpec((B, D), lambda j: (0, 0)),
            pl.BlockSpec((D, BN), lambda j: (0, j)),
            pl.BlockSpec((1, BN), lambda j: (0, j)),
        ],
        out_specs=pl.BlockSpec((B, BN), lambda j: (0, j)),
        out_shape=jax.ShapeDtypeStruct((B, NB * D), F32),
    )(z, lid_W, lid_b.reshape(1, NB * D))
    lot = lot2d.reshape(B * NB, D)
    lot3 = lot2d.reshape(B, NB, D)

    W1l, W1p = ed1_W[:D], ed1_W[D:D + NB]
    W2l, W2p = ed1_W[D + NB:2 * D + NB], ed1_W[2 * D + NB:]
    pos1 = jnp.tile(W1p, (GB, 1)) + ed1_b[None, :]
    pos2 = jnp.tile(W2p, (GB, 1))
    cmap2 = lambda *s: pl.BlockSpec(s, lambda j: (0,) * len(s))
    ar, adj, lep, lg, lu, lb, lm = pl.pallas_call(
        _fused_body,
        grid=(B // GB,),
        in_specs=[
            pl.BlockSpec((RB, D), lambda j: (j, 0)),
            pl.BlockSpec((GB, NB, D), lambda j: (j, 0, 0)),
            cmap2(D, D), cmap2(1, D), cmap2(D, 1), cmap2(1, 1),
            cmap2(D, D), cmap2(RB, D), cmap2(D, D), cmap2(RB, D),
            cmap2(1, D), cmap2(1, 1),
            cmap2(D, D), cmap2(NB, D), cmap2(D, D), cmap2(NB, D), cmap2(1, D),
            cmap2(D, D), cmap2(D, D), cmap2(1, D),
            cmap2(D, D), cmap2(D, D), cmap2(1, D),
            cmap2(D, 1), cmap2(1, 1), cmap2(D, NSEM), cmap2(1, NSEM),
            cmap2(D, 5), cmap2(1, 5), cmap2(D, 4), cmap2(1, 4),
            cmap2(D, 2), cmap2(1, 2),
        ],
        out_specs=[
            pl.BlockSpec((RB, 1), lambda j: (j, 0)),
            pl.BlockSpec((GB, NB, NB), lambda j: (j, 0, 0)),
            cmap2(B * NB, 1), cmap2(B * NB, 5), cmap2(B * NB, NSEM),
            cmap2(B * NB, 4), cmap2(B * NB, 2),
        ],
        out_shape=[
            jax.ShapeDtypeStruct((B * NB, 1), F32),
            jax.ShapeDtypeStruct((B, NB, NB), F32),
            jax.ShapeDtypeStruct((B * NB, 1), F32),
            jax.ShapeDtypeStruct((B * NB, 5), F32),
            jax.ShapeDtypeStruct((B * NB, NSEM), F32),
            jax.ShapeDtypeStruct((B * NB, 4), F32),
            jax.ShapeDtypeStruct((B * NB, 2), F32),
        ],
        scratch_shapes=[
            pltpu.VMEM((NB, NB), F32),
            pltpu.VMEM((NB, D), F32),
            pltpu.VMEM((NB, D), F32),
            pltpu.VMEM((NB, NB), F32),
        ],
    )(lot, lot3,
      bbd1_W, bbd1_b.reshape(1, D), bbd2_W, bbd2_b.reshape(1, 1),
      W1l, pos1, W2l, pos2, ed2_W.reshape(1, D), ed2_b.reshape(1, 1),
      mp1_W[:D], mp1_W[D:D + NB], mp1_W[D + NB:2 * D + NB], mp1_W[2 * D + NB:],
      mp1_b.reshape(1, D),
      mp2_W[:D], mp2_W[D:], mp2_b.reshape(1, D),
      mp3_W[:D], mp3_W[D:], mp3_b.reshape(1, D),
      ned_W, ned_b.reshape(1, 1), lh_W, lh_b.reshape(1, NSEM),
      bh_W, bh_b.reshape(1, 5), bdh_W, bdh_b.reshape(1, 4),
      mh_W, mh_b.reshape(1, 2))

    return (lep, lg, lu, lb, lm, adj, ar)


# speedup vs baseline: 1.7807x; 1.2453x over previous
import jax
import jax.numpy as jnp
from jax import lax
from jax.experimental import pallas as pl
from jax.experimental.pallas import tpu as pltpu

B = 64
NB = 30
NSEM = 11
D = 256
GB = 8
RB = GB * NB
F32 = jnp.float32


def _lot_body(z_ref, w_ref, b_ref, out_ref):
    acc = jnp.dot(z_ref[:, :], w_ref[:, :], preferred_element_type=F32)
    out_ref[:, :] = jnp.maximum(acc + b_ref[:, :], 0.0)


def _fused_body(lot2_ref, lot3_ref,
                bw1_ref, bb1_ref, bw2_ref, bb2_ref,
                w1_ref, p1_ref, w2_ref, p2_ref, edw_ref, edb_ref,
                w1il_ref, w1ip_ref, w1jl_ref, w1jp_ref, b1_ref,
                w2i_ref, w2j_ref, b2_ref, w3i_ref, w3j_ref, b3_ref,
                nedw_ref, nedb_ref, lhw_ref, lhb_ref, bhw_ref, bhb_ref,
                bdw_ref, bdb_ref, mhw_ref, mhb_ref,
                ar_ref, adj_ref, lep_ref, lg_ref, lu_ref, lb_ref, lm_ref,
                s_ref, nbr_ref, x30_ref, af0_ref):
    j = pl.program_id(0)
    lot = lot2_ref[:, :]

    h = jnp.maximum(jnp.dot(lot, bw1_ref[:, :], preferred_element_type=F32)
                    + bb1_ref[:, :], 0.0)
    ar_ref[:, :] = jnp.dot(h, bw2_ref[:, :], preferred_element_type=F32) + bb2_ref[:, :]

    w = edw_ref[:, :][None, :, :]
    eb = edb_ref[0, 0]
    for i in range(GB):
        li = lot3_ref[i, :, :]
        Ai = jnp.dot(li, w1_ref[:, :], preferred_element_type=F32) + p1_ref[:, :]
        Bi = jnp.dot(li, w2_ref[:, :], preferred_element_type=F32) + p2_ref[:, :]
        T = jnp.maximum(Ai[:, None, :] + Bi[None, :, :], 0.0)
        adj_ref[i, :, :] = jnp.sum(T * w, axis=-1) + eb

    adjb = jax.nn.sigmoid(adj_ref[:, :, :])
    adj_ref[:, :, :] = adjb
    af = (adjb >= 0.5).astype(F32)
    s_acc = jnp.sum(af, axis=0)
    nbr_acc = None
    for i in range(GB):
        nbr = lax.dot_general(af[i], lot3_ref[i, :, :], (((0,), (0,)), ((), ())),
                              preferred_element_type=F32)
        nbr_acc = nbr if nbr_acc is None else nbr_acc + nbr

    @pl.when(j == 0)
    def _init():
        s_ref[:, :] = s_acc
        nbr_ref[:, :] = nbr_acc
        x30_ref[:, :] = lot3_ref[0, :, :]
        af0_ref[:, :] = af[0]

    @pl.when(j > 0)
    def _acc():
        s_ref[:, :] = s_ref[:, :] + s_acc
        nbr_ref[:, :] = nbr_ref[:, :] + nbr_acc

    @pl.when(j == pl.num_programs(0) - 1)
    def _mp():
        S = s_ref[:, :]
        ones = jnp.ones((NB, 1), F32)
        deg = lax.dot_general(S, ones, (((0,), (0,)), ((), ())),
                              preferred_element_type=F32)
        invd = 1.0 / jnp.where(deg > 0, deg, 1.0)
        mask = deg > 0

        nbrm = nbr_ref[:, :] * invd
        sm = lax.dot_general(S, w1jp_ref[:, :], (((0,), (0,)), ((), ())),
                             preferred_element_type=F32) * invd
        out1 = (jnp.dot(x30_ref[:, :], w1il_ref[:, :], preferred_element_type=F32)
                + w1ip_ref[:, :]
                + jnp.dot(nbrm, w1jl_ref[:, :], preferred_element_type=F32)
                + sm + b1_ref[:, :])
        d = jnp.maximum(jnp.where(mask, out1, 0.0), 0.0)

        af0 = af0_ref[:, :]
        for wi_ref, wj_ref, bb_ref in ((w2i_ref, w2j_ref, b2_ref),
                                       (w3i_ref, w3j_ref, b3_ref)):
            nbr2 = lax.dot_general(af0, d, (((0,), (0,)), ((), ())),
                                   preferred_element_type=F32) * invd
            out = (jnp.dot(d, wi_ref[:, :], preferred_element_type=F32)
                   + jnp.dot(nbr2, wj_ref[:, :], preferred_element_type=F32)
                   + bb_ref[:, :])
            d = jnp.maximum(jnp.where(mask, out, 0.0), 0.0)

        def head(w_ref, b_ref):
            return jnp.dot(d, w_ref[:, :], preferred_element_type=F32) + b_ref[:, :]

        def softmax(x):
            m = jnp.max(x, axis=-1, keepdims=True)
            e = jnp.exp(x - m)
            return e / jnp.sum(e, axis=-1, keepdims=True)

        rest = B * NB - NB
        lep_ref[:NB, :] = jax.nn.sigmoid(head(nedw_ref, nedb_ref))
        lep_ref[NB:, :] = jnp.broadcast_to(jax.nn.sigmoid(nedb_ref[:, :]), (rest, 1))
        lu_ref[:NB, :] = softmax(head(lhw_ref, lhb_ref))
        lu_ref[NB:, :] = jnp.broadcast_to(softmax(lhb_ref[:, :]), (rest, NSEM))
        lg_ref[:NB, :] = head(bhw_ref, bhb_ref)
        lg_ref[NB:, :] = jnp.broadcast_to(bhb_ref[:, :], (rest, 5))
        lb_ref[:NB, :] = head(bdw_ref, bdb_ref)
        lb_ref[NB:, :] = jnp.broadcast_to(bdb_ref[:, :], (rest, 4))
        lm_ref[:NB, :] = head(mhw_ref, mhb_ref)
        lm_ref[NB:, :] = jnp.broadcast_to(mhb_ref[:, :], (rest, 2))


def kernel(z, lid_W, lid_b, bbd1_W, bbd1_b, bbd2_W, bbd2_b, ed1_W, ed1_b,
           ed2_W, ed2_b, mp1_W, mp1_b, mp2_W, mp2_b, mp3_W, mp3_b,
           ned_W, ned_b, lh_W, lh_b, bh_W, bh_b, bdh_W, bdh_b, mh_W, mh_b):
    NBLK = 6
    BN = (NB * D) // NBLK
    lot2d = pl.pallas_call(
        _lot_body,
        grid=(NBLK,),
        in_specs=[
            pl.BlockSpec((B, D), lambda j: (0, 0)),
            pl.BlockSpec((D, BN), lambda j: (0, j)),
            pl.BlockSpec((1, BN), lambda j: (0, j)),
        ],
        out_specs=pl.BlockSpec((B, BN), lambda j: (0, j)),
        out_shape=jax.ShapeDtypeStruct((B, NB * D), F32),
    )(z, lid_W, lid_b.reshape(1, NB * D))
    lot = lot2d.reshape(B * NB, D)
    lot3 = lot2d.reshape(B, NB, D)

    W1l, W1p = ed1_W[:D], ed1_W[D:D + NB]
    W2l, W2p = ed1_W[D + NB:2 * D + NB], ed1_W[2 * D + NB:]
    pos1 = W1p + ed1_b[None, :]
    pos2 = W2p
    cmap2 = lambda *s: pl.BlockSpec(s, lambda j: (0,) * len(s))
    ar, adj, lep, lg, lu, lb, lm = pl.pallas_call(
        _fused_body,
        grid=(B // GB,),
        in_specs=[
            pl.BlockSpec((RB, D), lambda j: (j, 0)),
            pl.BlockSpec((GB, NB, D), lambda j: (j, 0, 0)),
            cmap2(D, D), cmap2(1, D), cmap2(D, 1), cmap2(1, 1),
            cmap2(D, D), cmap2(NB, D), cmap2(D, D), cmap2(NB, D),
            cmap2(1, D), cmap2(1, 1),
            cmap2(D, D), cmap2(NB, D), cmap2(D, D), cmap2(NB, D), cmap2(1, D),
            cmap2(D, D), cmap2(D, D), cmap2(1, D),
            cmap2(D, D), cmap2(D, D), cmap2(1, D),
            cmap2(D, 1), cmap2(1, 1), cmap2(D, NSEM), cmap2(1, NSEM),
            cmap2(D, 5), cmap2(1, 5), cmap2(D, 4), cmap2(1, 4),
            cmap2(D, 2), cmap2(1, 2),
        ],
        out_specs=[
            pl.BlockSpec((RB, 1), lambda j: (j, 0)),
            pl.BlockSpec((GB, NB, NB), lambda j: (j, 0, 0)),
            cmap2(B * NB, 1), cmap2(B * NB, 5), cmap2(B * NB, NSEM),
            cmap2(B * NB, 4), cmap2(B * NB, 2),
        ],
        out_shape=[
            jax.ShapeDtypeStruct((B * NB, 1), F32),
            jax.ShapeDtypeStruct((B, NB, NB), F32),
            jax.ShapeDtypeStruct((B * NB, 1), F32),
            jax.ShapeDtypeStruct((B * NB, 5), F32),
            jax.ShapeDtypeStruct((B * NB, NSEM), F32),
            jax.ShapeDtypeStruct((B * NB, 4), F32),
            jax.ShapeDtypeStruct((B * NB, 2), F32),
        ],
        scratch_shapes=[
            pltpu.VMEM((NB, NB), F32),
            pltpu.VMEM((NB, D), F32),
            pltpu.VMEM((NB, D), F32),
            pltpu.VMEM((NB, NB), F32),
        ],
    )(lot, lot3,
      bbd1_W, bbd1_b.reshape(1, D), bbd2_W, bbd2_b.reshape(1, 1),
      W1l, pos1, W2l, pos2, ed2_W.reshape(1, D), ed2_b.reshape(1, 1),
      mp1_W[:D], mp1_W[D:D + NB], mp1_W[D + NB:2 * D + NB], mp1_W[2 * D + NB:],
      mp1_b.reshape(1, D),
      mp2_W[:D], mp2_W[D:], mp2_b.reshape(1, D),
      mp3_W[:D], mp3_W[D:], mp3_b.reshape(1, D),
      ned_W, ned_b.reshape(1, 1), lh_W, lh_b.reshape(1, NSEM),
      bh_W, bh_b.reshape(1, 5), bdh_W, bdh_b.reshape(1, 4),
      mh_W, mh_b.reshape(1, 2))

    return (lep, lg, lu, lb, lm, adj, ar)


# speedup vs baseline: 1.8982x; 1.0660x over previous
import jax
import jax.numpy as jnp
from jax import lax
from jax.experimental import pallas as pl
from jax.experimental.pallas import tpu as pltpu

B = 64
NB = 30
NSEM = 11
D = 256
GB = 16
RB = GB * NB
F32 = jnp.float32


def _lot_body(z_ref, w_ref, b_ref, out_ref):
    acc = jnp.dot(z_ref[:, :], w_ref[:, :], preferred_element_type=F32)
    out_ref[:, :] = jnp.maximum(acc + b_ref[:, :], 0.0)


def _fused_body(lot2_ref, lot3_ref,
                bw1_ref, bb1_ref, bw2_ref, bb2_ref,
                w1_ref, p1_ref, w2_ref, p2_ref, edw_ref, edb_ref,
                w1il_ref, w1ip_ref, w1jl_ref, w1jp_ref, b1_ref,
                w2i_ref, w2j_ref, b2_ref, w3i_ref, w3j_ref, b3_ref,
                nedw_ref, nedb_ref, lhw_ref, lhb_ref, bhw_ref, bhb_ref,
                bdw_ref, bdb_ref, mhw_ref, mhb_ref,
                ar_ref, adj_ref, lep_ref, lg_ref, lu_ref, lb_ref, lm_ref,
                s_ref, nbr_ref, x30_ref, af0_ref):
    j = pl.program_id(0)
    lot = lot2_ref[:, :]

    h = jnp.maximum(jnp.dot(lot, bw1_ref[:, :], preferred_element_type=F32)
                    + bb1_ref[:, :], 0.0)
    ar_ref[:, :] = jnp.dot(h, bw2_ref[:, :], preferred_element_type=F32) + bb2_ref[:, :]

    w = edw_ref[:, :][None, :, :]
    eb = edb_ref[0, 0]
    for i in range(GB):
        li = lot3_ref[i, :, :]
        Ai = jnp.dot(li, w1_ref[:, :], preferred_element_type=F32) + p1_ref[:, :]
        Bi = jnp.dot(li, w2_ref[:, :], preferred_element_type=F32) + p2_ref[:, :]
        T = jnp.maximum(Ai[:, None, :] + Bi[None, :, :], 0.0)
        adj_ref[i, :, :] = jnp.sum(T * w, axis=-1) + eb

    adjb = jax.nn.sigmoid(adj_ref[:, :, :])
    adj_ref[:, :, :] = adjb
    af = (adjb >= 0.5).astype(F32)
    s_acc = jnp.sum(af, axis=0)
    nbr_acc = None
    for i in range(GB):
        nbr = lax.dot_general(af[i], lot3_ref[i, :, :], (((0,), (0,)), ((), ())),
                              preferred_element_type=F32)
        nbr_acc = nbr if nbr_acc is None else nbr_acc + nbr

    @pl.when(j == 0)
    def _init():
        s_ref[:, :] = s_acc
        nbr_ref[:, :] = nbr_acc
        x30_ref[:, :] = lot3_ref[0, :, :]
        af0_ref[:, :] = af[0]

    @pl.when(j > 0)
    def _acc():
        s_ref[:, :] = s_ref[:, :] + s_acc
        nbr_ref[:, :] = nbr_ref[:, :] + nbr_acc

    @pl.when(j == pl.num_programs(0) - 1)
    def _mp():
        S = s_ref[:, :]
        ones = jnp.ones((NB, 1), F32)
        deg = lax.dot_general(S, ones, (((0,), (0,)), ((), ())),
                              preferred_element_type=F32)
        invd = 1.0 / jnp.where(deg > 0, deg, 1.0)
        mask = deg > 0

        nbrm = nbr_ref[:, :] * invd
        sm = lax.dot_general(S, w1jp_ref[:, :], (((0,), (0,)), ((), ())),
                             preferred_element_type=F32) * invd
        out1 = (jnp.dot(x30_ref[:, :], w1il_ref[:, :], preferred_element_type=F32)
                + w1ip_ref[:, :]
                + jnp.dot(nbrm, w1jl_ref[:, :], preferred_element_type=F32)
                + sm + b1_ref[:, :])
        d = jnp.maximum(jnp.where(mask, out1, 0.0), 0.0)

        af0 = af0_ref[:, :]
        for wi_ref, wj_ref, bb_ref in ((w2i_ref, w2j_ref, b2_ref),
                                       (w3i_ref, w3j_ref, b3_ref)):
            nbr2 = lax.dot_general(af0, d, (((0,), (0,)), ((), ())),
                                   preferred_element_type=F32) * invd
            out = (jnp.dot(d, wi_ref[:, :], preferred_element_type=F32)
                   + jnp.dot(nbr2, wj_ref[:, :], preferred_element_type=F32)
                   + bb_ref[:, :])
            d = jnp.maximum(jnp.where(mask, out, 0.0), 0.0)

        def head(w_ref, b_ref):
            return jnp.dot(d, w_ref[:, :], preferred_element_type=F32) + b_ref[:, :]

        def softmax(x):
            m = jnp.max(x, axis=-1, keepdims=True)
            e = jnp.exp(x - m)
            return e / jnp.sum(e, axis=-1, keepdims=True)

        rest = B * NB - NB
        lep_ref[:NB, :] = jax.nn.sigmoid(head(nedw_ref, nedb_ref))
        lep_ref[NB:, :] = jnp.broadcast_to(jax.nn.sigmoid(nedb_ref[:, :]), (rest, 1))
        lu_ref[:NB, :] = softmax(head(lhw_ref, lhb_ref))
        lu_ref[NB:, :] = jnp.broadcast_to(softmax(lhb_ref[:, :]), (rest, NSEM))
        lg_ref[:NB, :] = head(bhw_ref, bhb_ref)
        lg_ref[NB:, :] = jnp.broadcast_to(bhb_ref[:, :], (rest, 5))
        lb_ref[:NB, :] = head(bdw_ref, bdb_ref)
        lb_ref[NB:, :] = jnp.broadcast_to(bdb_ref[:, :], (rest, 4))
        lm_ref[:NB, :] = head(mhw_ref, mhb_ref)
        lm_ref[NB:, :] = jnp.broadcast_to(mhb_ref[:, :], (rest, 2))


def kernel(z, lid_W, lid_b, bbd1_W, bbd1_b, bbd2_W, bbd2_b, ed1_W, ed1_b,
           ed2_W, ed2_b, mp1_W, mp1_b, mp2_W, mp2_b, mp3_W, mp3_b,
           ned_W, ned_b, lh_W, lh_b, bh_W, bh_b, bdh_W, bdh_b, mh_W, mh_b):
    NBLK = 2
    BN = (NB * D) // NBLK
    lot2d = pl.pallas_call(
        _lot_body,
        grid=(NBLK,),
        in_specs=[
            pl.BlockSpec((B, D), lambda j: (0, 0)),
            pl.BlockSpec((D, BN), lambda j: (0, j)),
            pl.BlockSpec((1, BN), lambda j: (0, j)),
        ],
        out_specs=pl.BlockSpec((B, BN), lambda j: (0, j)),
        out_shape=jax.ShapeDtypeStruct((B, NB * D), F32),
    )(z, lid_W, lid_b.reshape(1, NB * D))
    lot = lot2d.reshape(B * NB, D)
    lot3 = lot2d.reshape(B, NB, D)

    W1l, W1p = ed1_W[:D], ed1_W[D:D + NB]
    W2l, W2p = ed1_W[D + NB:2 * D + NB], ed1_W[2 * D + NB:]
    pos1 = W1p + ed1_b[None, :]
    pos2 = W2p
    cmap2 = lambda *s: pl.BlockSpec(s, lambda j: (0,) * len(s))
    ar, adj, lep, lg, lu, lb, lm = pl.pallas_call(
        _fused_body,
        grid=(B // GB,),
        in_specs=[
            pl.BlockSpec((RB, D), lambda j: (j, 0)),
            pl.BlockSpec((GB, NB, D), lambda j: (j, 0, 0)),
            cmap2(D, D), cmap2(1, D), cmap2(D, 1), cmap2(1, 1),
            cmap2(D, D), cmap2(NB, D), cmap2(D, D), cmap2(NB, D),
            cmap2(1, D), cmap2(1, 1),
            cmap2(D, D), cmap2(NB, D), cmap2(D, D), cmap2(NB, D), cmap2(1, D),
            cmap2(D, D), cmap2(D, D), cmap2(1, D),
            cmap2(D, D), cmap2(D, D), cmap2(1, D),
            cmap2(D, 1), cmap2(1, 1), cmap2(D, NSEM), cmap2(1, NSEM),
            cmap2(D, 5), cmap2(1, 5), cmap2(D, 4), cmap2(1, 4),
            cmap2(D, 2), cmap2(1, 2),
        ],
        out_specs=[
            pl.BlockSpec((RB, 1), lambda j: (j, 0)),
            pl.BlockSpec((GB, NB, NB), lambda j: (j, 0, 0)),
            cmap2(B * NB, 1), cmap2(B * NB, 5), cmap2(B * NB, NSEM),
            cmap2(B * NB, 4), cmap2(B * NB, 2),
        ],
        out_shape=[
            jax.ShapeDtypeStruct((B * NB, 1), F32),
            jax.ShapeDtypeStruct((B, NB, NB), F32),
            jax.ShapeDtypeStruct((B * NB, 1), F32),
            jax.ShapeDtypeStruct((B * NB, 5), F32),
            jax.ShapeDtypeStruct((B * NB, NSEM), F32),
            jax.ShapeDtypeStruct((B * NB, 4), F32),
            jax.ShapeDtypeStruct((B * NB, 2), F32),
        ],
        scratch_shapes=[
            pltpu.VMEM((NB, NB), F32),
            pltpu.VMEM((NB, D), F32),
            pltpu.VMEM((NB, D), F32),
            pltpu.VMEM((NB, NB), F32),
        ],
    )(lot, lot3,
      bbd1_W, bbd1_b.reshape(1, D), bbd2_W, bbd2_b.reshape(1, 1),
      W1l, pos1, W2l, pos2, ed2_W.reshape(1, D), ed2_b.reshape(1, 1),
      mp1_W[:D], mp1_W[D:D + NB], mp1_W[D + NB:2 * D + NB], mp1_W[2 * D + NB:],
      mp1_b.reshape(1, D),
      mp2_W[:D], mp2_W[D:], mp2_b.reshape(1, D),
      mp3_W[:D], mp3_W[D:], mp3_b.reshape(1, D),
      ned_W, ned_b.reshape(1, 1), lh_W, lh_b.reshape(1, NSEM),
      bh_W, bh_b.reshape(1, 5), bdh_W, bdh_b.reshape(1, 4),
      mh_W, mh_b.reshape(1, 2))

    return (lep, lg, lu, lb, lm, adj, ar)
